# baseline (speedup 1.0000x reference)
import functools

import jax
import jax.numpy as jnp
from jax import lax
from jax.experimental import pallas as pl
from jax.experimental.pallas import tpu as pltpu
from jax.experimental.pallas import tpu_sc as plsc

N = 10000
IN_DIM = 128
HIDDEN = 128
HEADS = 8
DH = 16
K = 7
M = 5
N_UP = 2500
N_DOWN = N - N_UP

NPAD = 10112
BQ = 256
BQK = 512
NUP_PAD = 2560
BQ2 = 512
BN = 1000

_INF = float('inf')


def _preamble_body(h_ref, wc_ref, bc_ref, we_ref, be_ref, wq_ref, bq_ref,
                   wkv_ref, bkv_ref, sl_ref, hh_ref, q_ref, kv_ref):
    h = h_ref[...]
    sl_ref[...] = jnp.dot(h, wc_ref[...], preferred_element_type=jnp.float32) + bc_ref[...]
    hh = jnp.dot(h, we_ref[...], preferred_element_type=jnp.float32) + be_ref[...]
    hh_ref[...] = hh
    q_ref[...] = jnp.dot(hh, wq_ref[...], preferred_element_type=jnp.float32) + bq_ref[...]
    kv_ref[...] = jnp.dot(hh, wkv_ref[...], preferred_element_type=jnp.float32) + bkv_ref[...]


def _preamble(h, wc8, bc8, we, be, wq, bq, wkv, bkv):
    grid = (N // BN,)
    row = pl.BlockSpec((BN, IN_DIM), lambda i: (i, 0))
    full = lambda r, c: pl.BlockSpec((r, c), lambda i: (0, 0))
    return pl.pallas_call(
        _preamble_body,
        grid=grid,
        in_specs=[row, full(IN_DIM, 8), full(1, 8), full(IN_DIM, HIDDEN),
                  full(1, HIDDEN), full(HIDDEN, HIDDEN), full(1, HIDDEN),
                  full(HIDDEN, 2 * HIDDEN), full(1, 2 * HIDDEN)],
        out_specs=[pl.BlockSpec((BN, 8), lambda i: (i, 0)),
                   pl.BlockSpec((BN, HIDDEN), lambda i: (i, 0)),
                   pl.BlockSpec((BN, HIDDEN), lambda i: (i, 0)),
                   pl.BlockSpec((BN, 2 * HIDDEN), lambda i: (i, 0))],
        out_shape=[jax.ShapeDtypeStruct((N, 8), jnp.float32),
                   jax.ShapeDtypeStruct((N, HIDDEN), jnp.float32),
                   jax.ShapeDtypeStruct((N, HIDDEN), jnp.float32),
                   jax.ShapeDtypeStruct((N, 2 * HIDDEN), jnp.float32)],
    )(h, wc8, bc8, we, be, wq, bq, wkv, bkv)


W_KNN = 640
S_KNN = 16
NPAD2 = W_KNN * S_KNN
_PACK = 1024
_PACK_BITS = 10


def _knn_body(sl_ref, slt_ref, t_ref, out_ref):
    pid = pl.program_id(0)
    qx = sl_ref[:, 0:1]
    qy = sl_ref[:, 1:2]
    qz = sl_ref[:, 2:3]
    jcol = lax.broadcasted_iota(jnp.int32, (BQK, W_KNN), 1)
    row = pid * BQK + lax.broadcasted_iota(jnp.int32, (BQK, W_KNN), 0)
    mbest = jnp.full((BQK, W_KNN), _INF, jnp.float32)
    abest = jnp.zeros((BQK, W_KNN), jnp.int32)
    for m in range(S_KNN):
        dx = qx - slt_ref[0:1, m * W_KNN:(m + 1) * W_KNN]
        dy = qy - slt_ref[1:2, m * W_KNN:(m + 1) * W_KNN]
        dz = qz - slt_ref[2:3, m * W_KNN:(m + 1) * W_KNN]
        d = dx * dx + dy * dy + dz * dz
        d = jnp.where(m * W_KNN + jcol == row, _INF, d)
        better = d < mbest
        mbest = jnp.where(better, d, mbest)
        abest = jnp.where(better, m * _PACK + jcol, abest)

    rowc = pid * BQK + lax.broadcasted_iota(jnp.int32, (BQK, 1), 0)
    found = []
    for t in range(K):
        mv = jnp.min(mbest, axis=1, keepdims=True)
        p = jnp.min(jnp.where(mbest == mv, abest, 2 ** 30), axis=1, keepdims=True)
        jstar = jnp.bitwise_and(p, _PACK - 1)
        mstar = lax.shift_right_logical(p, _PACK_BITS)
        gsel = mstar * W_KNN + jstar
        out_ref[:, t:t + 1] = gsel
        found.append(gsel)
        if t == K - 1:
            break
        onehot = jcol == jstar
        cand = jnp.dot(onehot.astype(jnp.float32), t_ref[...],
                       preferred_element_type=jnp.float32)
        ds = []
        for m in range(S_KNN):
            dx = qx - cand[:, 4 * m:4 * m + 1]
            dy = qy - cand[:, 4 * m + 1:4 * m + 2]
            dz = qz - cand[:, 4 * m + 2:4 * m + 3]
            dm = dx * dx + dy * dy + dz * dz
            gc = m * W_KNN + jstar
            bad = gc == rowc
            for f in found:
                bad = bad | (gc == f)
            ds.append(jnp.where(bad, _INF, dm))
        dall = jnp.concatenate(ds, axis=1)
        nm = jnp.min(dall, axis=1, keepdims=True)
        mi = jnp.min(jnp.where(dall == nm,
                               lax.broadcasted_iota(jnp.int32, (BQK, S_KNN), 1),
                               S_KNN), axis=1, keepdims=True)
        na = mi * _PACK + jstar
        mbest = jnp.where(onehot, jnp.broadcast_to(nm, (BQK, W_KNN)), mbest)
        abest = jnp.where(onehot, jnp.broadcast_to(na, (BQK, W_KNN)), abest)
    out_ref[:, K:K + 1] = jnp.zeros((BQK, 1), jnp.int32)


def _knn(sl8, slt, tbl):
    grid = (pl.cdiv(N, BQK),)
    return pl.pallas_call(
        _knn_body,
        grid=grid,
        in_specs=[pl.BlockSpec((BQK, 8), lambda i: (i, 0)),
                  pl.BlockSpec((8, NPAD2), lambda i: (0, 0)),
                  pl.BlockSpec((W_KNN, 4 * S_KNN), lambda i: (0, 0))],
        out_specs=pl.BlockSpec((BQK, 8), lambda i: (i, 0)),
        out_shape=jax.ShapeDtypeStruct((N, 8), jnp.int32),
    )(sl8, slt, tbl)


def _head_ones():
    r = lax.broadcasted_iota(jnp.int32, (HIDDEN, HEADS), 0)
    c = lax.broadcasted_iota(jnp.int32, (HIDDEN, HEADS), 1)
    return (r // DH == c).astype(jnp.float32)


def _ln(x, s, b):
    m = jnp.mean(x, axis=-1, keepdims=True)
    v = jnp.mean((x - m) ** 2, axis=-1, keepdims=True)
    return (x - m) / jnp.sqrt(v + 1e-5) * s + b


def _attn_core(hh_ref, q_ref, kvg_ref, wo_ref, bo_ref, l1s_ref, l1b_ref,
               w1_ref, b1_ref, w2_ref, b2_ref, l2s_ref, l2b_ref):
    oh = _head_ones()
    q = q_ref[...]
    ss = []
    for kk in range(K):
        kblk = kvg_ref[kk, :, 0:128]
        ss.append(jnp.dot(kblk * q, oh, preferred_element_type=jnp.float32) * 0.25)
    smax = ss[0]
    for kk in range(1, K):
        smax = jnp.maximum(smax, ss[kk])
    exs = [jnp.exp(s - smax) for s in ss]
    denom = exs[0]
    for kk in range(1, K):
        denom = denom + exs[kk]
    denom = denom + 1e-6
    wv = None
    for kk in range(K):
        ex128 = jnp.dot(exs[kk], oh.T, preferred_element_type=jnp.float32)
        vblk = kvg_ref[kk, :, 128:256]
        wv = ex128 * vblk if wv is None else wv + ex128 * vblk
    attn = wv / jnp.dot(denom, oh.T, preferred_element_type=jnp.float32)
    h2 = hh_ref[...] + jnp.dot(attn, wo_ref[...], preferred_element_type=jnp.float32) + bo_ref[...]
    h2 = _ln(h2, l1s_ref[...], l1b_ref[...])
    ff = jnp.dot(jax.nn.relu(jnp.dot(h2, w1_ref[...], preferred_element_type=jnp.float32) + b1_ref[...]),
                 w2_ref[...], preferred_element_type=jnp.float32) + b2_ref[...]
    return _ln(h2 + ff, l2s_ref[...], l2b_ref[...])


def _attn1_body(hh_ref, q_ref, kvg_ref, wo_ref, bo_ref, l1s_ref, l1b_ref,
                w1_ref, b1_ref, w2_ref, b2_ref, l2s_ref, l2b_ref,
                wq2_ref, bq2_ref, wkv2_ref, bkv2_ref,
                h3_ref, q2_ref, kv2_ref):
    h3 = _attn_core(hh_ref, q_ref, kvg_ref, wo_ref, bo_ref, l1s_ref, l1b_ref,
                    w1_ref, b1_ref, w2_ref, b2_ref, l2s_ref, l2b_ref)
    h3_ref[...] = h3
    q2_ref[...] = jnp.dot(h3, wq2_ref[...], preferred_element_type=jnp.float32) + bq2_ref[...]
    kv2_ref[...] = jnp.dot(h3, wkv2_ref[...], preferred_element_type=jnp.float32) + bkv2_ref[...]


def _attn2_body(hh_ref, q_ref, kvg_ref, wo_ref, bo_ref, l1s_ref, l1b_ref,
                w1_ref, b1_ref, w2_ref, b2_ref, l2s_ref, l2b_ref,
                sl_ref, wa_ref, wac_ref, ba_ref, feats_ref):
    h3 = _attn_core(hh_ref, q_ref, kvg_ref, wo_ref, bo_ref, l1s_ref, l1b_ref,
                    w1_ref, b1_ref, w2_ref, b2_ref, l2s_ref, l2b_ref)
    feats_ref[...] = (jnp.dot(h3, wa_ref[...], preferred_element_type=jnp.float32)
                      + jnp.dot(sl_ref[...], wac_ref[...], preferred_element_type=jnp.float32)
                      + ba_ref[...])


BNA = 1024
NP_E = 10240


def _attn_specs():
    row128 = pl.BlockSpec((BNA, HIDDEN), lambda i: (i, 0))
    rowkvg = pl.BlockSpec((K, BNA, 256), lambda i: (0, i, 0))
    full = lambda r, c: pl.BlockSpec((r, c), lambda i: (0, 0))
    w = [full(HIDDEN, HIDDEN), full(1, HIDDEN), full(1, HIDDEN), full(1, HIDDEN),
         full(HIDDEN, 2 * HIDDEN), full(1, 2 * HIDDEN), full(2 * HIDDEN, HIDDEN),
         full(1, HIDDEN), full(1, HIDDEN), full(1, HIDDEN)]
    return [row128, row128, rowkvg] + w, full


def _attn1(hh, q, kvg, lw, wq2, bq2, wkv2, bkv2):
    base, full = _attn_specs()
    in_specs = base + [full(HIDDEN, HIDDEN), full(1, HIDDEN),
                       full(HIDDEN, 2 * HIDDEN), full(1, 2 * HIDDEN)]
    row128 = pl.BlockSpec((BNA, HIDDEN), lambda i: (i, 0))
    return pl.pallas_call(
        _attn1_body,
        grid=(NP_E // BNA,),
        in_specs=in_specs,
        out_specs=[row128, row128, pl.BlockSpec((BNA, 2 * HIDDEN), lambda i: (i, 0))],
        out_shape=[jax.ShapeDtypeStruct((N, HIDDEN), jnp.float32),
                   jax.ShapeDtypeStruct((N, HIDDEN), jnp.float32),
                   jax.ShapeDtypeStruct((N, 2 * HIDDEN), jnp.float32)],
    )(hh, q, kvg, *lw, wq2, bq2, wkv2, bkv2)


def _attn2(hh, q, kvg, lw, sl8, wa, wac8, ba):
    base, full = _attn_specs()
    in_specs = base + [pl.BlockSpec((BNA, 8), lambda i: (i, 0)),
                       full(HIDDEN, HIDDEN), full(8, HIDDEN), full(1, HIDDEN)]
    return pl.pallas_call(
        _attn2_body,
        grid=(NP_E // BNA,),
        in_specs=in_specs,
        out_specs=pl.BlockSpec((BNA, HIDDEN), lambda i: (i, 0)),
        out_shape=jax.ShapeDtypeStruct((N, HIDDEN), jnp.float32),
    )(hh, q, kvg, *lw, sl8, wa, wac8, ba)


def _rank_body(s8_ref, st_ref, out_ref):
    pid = pl.program_id(0)
    si = s8_ref[:, 0:1]
    srow = st_ref[0:1, :]
    col = lax.broadcasted_iota(jnp.int32, (BQ, NPAD), 1)
    row = pid * BQ + lax.broadcasted_iota(jnp.int32, (BQ, NPAD), 0)
    gt = (srow > si).astype(jnp.int32)
    eq = ((srow == si) & (col > row)).astype(jnp.int32)
    rank = jnp.sum(gt + eq, axis=1, keepdims=True)
    out_ref[...] = jnp.broadcast_to((rank < N_UP).astype(jnp.int32), (BQ, 8))


def _rank(scores8, scorest):
    return pl.pallas_call(
        _rank_body,
        grid=(pl.cdiv(N, BQ),),
        in_specs=[pl.BlockSpec((BQ, 8), lambda i: (i, 0)),
                  pl.BlockSpec((1, NPAD), lambda i: (0, 0))],
        out_specs=pl.BlockSpec((BQ, 8), lambda i: (i, 0)),
        out_shape=jax.ShapeDtypeStruct((N, 8), jnp.int32),
    )(scores8, scorest)


ND_PAD = 7680


def _dknn_body(dc_ref, upt_ref, out_ref, dist_ref):
    qx = dc_ref[:, 0:1]
    qy = dc_ref[:, 1:2]
    qz = dc_ref[:, 2:3]
    dx = qx - upt_ref[0:1, :]
    dy = qy - upt_ref[1:2, :]
    dz = qz - upt_ref[2:3, :]
    dist_ref[...] = dx * dx + dy * dy + dz * dz
    col = lax.broadcasted_iota(jnp.int32, (BQ2, NUP_PAD), 1)
    for t in range(M):
        d = dist_ref[...]
        m = jnp.min(d, axis=1, keepdims=True)
        idx = jnp.min(jnp.where(d == m, col, NUP_PAD), axis=1, keepdims=True)
        out_ref[:, t:t + 1] = idx
        dist_ref[...] = jnp.where(col == idx, _INF, d)
    out_ref[:, M:M + 3] = jnp.zeros((BQ2, 3), jnp.int32)


def _dknn(downc, upt):
    return pl.pallas_call(
        _dknn_body,
        grid=(ND_PAD // BQ2,),
        in_specs=[pl.BlockSpec((BQ2, 8), lambda i: (i, 0)),
                  pl.BlockSpec((8, NUP_PAD), lambda i: (0, 0))],
        out_specs=pl.BlockSpec((BQ2, 8), lambda i: (i, 0)),
        out_shape=jax.ShapeDtypeStruct((ND_PAD, 8), jnp.int32),
        scratch_shapes=[pltpu.VMEM((BQ2, NUP_PAD), jnp.float32)],
    )(downc, upt)


_NW = 32


def _sc_gather(table, idx, chunk):
    b = idx.shape[0]
    d = table.shape[1]
    b_per_w = b // _NW
    nchunks = b_per_w // chunk
    mesh = plsc.VectorSubcoreMesh(core_axis_name="c", subcore_axis_name="s")

    @functools.partial(
        pl.kernel, mesh=mesh,
        out_type=jax.ShapeDtypeStruct((b, d), jnp.float32),
        scratch_types=[pltpu.VMEM((chunk,), jnp.int32),
                       pltpu.VMEM((chunk, d), jnp.float32),
                       pltpu.SemaphoreType.DMA],
    )
    def k(table_hbm, idx_hbm, out_hbm, idx_v, rows_v, sem):
        wid = lax.axis_index("s") * 2 + lax.axis_index("c")
        base = wid * b_per_w

        def body(ci, _):
            off = base + ci * chunk
            pltpu.sync_copy(idx_hbm.at[pl.ds(off, chunk)], idx_v)
            pltpu.async_copy(table_hbm.at[idx_v], rows_v, sem).wait()
            pltpu.sync_copy(rows_v, out_hbm.at[pl.ds(off, chunk)])
            return 0

        lax.fori_loop(0, nchunks, body, 0)

    return k(table, idx)


def _sc_partition(mask_i32):
    mesh = plsc.VectorSubcoreMesh(core_axis_name="c", subcore_axis_name="s")

    @functools.partial(
        pl.kernel, mesh=mesh,
        compiler_params=pltpu.CompilerParams(needs_layout_passes=False),
        out_type=[jax.ShapeDtypeStruct((2512,), jnp.int32),
                  jax.ShapeDtypeStruct((7504,), jnp.int32)],
        scratch_types=[pltpu.VMEM((N,), jnp.int32),
                       pltpu.VMEM((2512,), jnp.int32),
                       pltpu.VMEM((7504,), jnp.int32)],
    )
    def k(mask_hbm, up_hbm, down_hbm, mask_v, up_v, down_v):
        wid = lax.axis_index("s") * 2 + lax.axis_index("c")

        def compact(target, out_v):
            pltpu.sync_copy(mask_hbm, mask_v)

            def body(ci, off):
                v = mask_v[pl.ds(ci * 16, 16)]
                want = v == jnp.full((16,), target, jnp.int32)
                ids = lax.iota(jnp.int32, 16) + jnp.full((16,), ci * 16, jnp.int32)
                csum = jnp.cumsum(want.astype(jnp.int32))
                pos = csum + jnp.full((16,), off - 1, jnp.int32)
                plsc.store_scatter(out_v, [pos], ids, mask=want)
                return off + jnp.sum(want.astype(jnp.int32))

            lax.fori_loop(0, N // 16, body, jnp.int32(0))

        @pl.when(wid == 0)
        def _():
            compact(1, up_v)
            pltpu.sync_copy(up_v, up_hbm)

        @pl.when(wid == 1)
        def _():
            compact(0, down_v)
            pltpu.sync_copy(down_v, down_hbm)

    return k(mask_i32)


def kernel(h, c, scores, object_, params):
    p = params
    l1, l2 = p['layers'][0], p['layers'][1]

    wc8 = jnp.pad(p['W_coord'], ((0, 0), (0, 5)))
    bc8 = jnp.pad(p['b_coord'], (0, 5)).reshape(1, 8)
    row = lambda x: x.reshape(1, -1)
    wkv1 = jnp.concatenate([l1['Wk'], l1['Wv']], axis=1)
    bkv1 = jnp.concatenate([l1['bk'], l1['bv']]).reshape(1, -1)
    wkv2 = jnp.concatenate([l2['Wk'], l2['Wv']], axis=1)
    bkv2 = jnp.concatenate([l2['bk'], l2['bv']]).reshape(1, -1)

    sl8, hh0, q1, kv1 = _preamble(h, wc8, bc8, p['W_emb'], row(p['b_emb']),
                                  l1['Wq'], row(l1['bq']), wkv1, bkv1)

    slt = jnp.pad(sl8.T[:3], ((0, 5), (0, NPAD2 - N)), constant_values=1e30)
    tbl = jnp.pad(slt[:3].reshape(3, S_KNN, W_KNN).transpose(2, 1, 0),
                  ((0, 0), (0, 0), (0, 1))).reshape(W_KNN, 4 * S_KNN)
    knn = _knn(sl8, slt, tbl)
    src = knn[:, :K]

    lw = lambda l: (l['Wo'], row(l['bo']), row(l['ln1_s']), row(l['ln1_b']),
                    l['W1'], row(l['b1']), l['W2'], row(l['b2']),
                    row(l['ln2_s']), row(l['ln2_b']))

    eidx = jnp.pad(src.T, ((0, 0), (0, NP_E - N))).reshape(-1)

    kvg1 = _sc_gather(kv1, eidx, 448).reshape(K, NP_E, 256)
    h1, q2, kv2 = _attn1(hh0, q1, kvg1, lw(l1), l2['Wq'], row(l2['bq']), wkv2, bkv2)

    wa = p['W_att'][:HIDDEN]
    wac8 = jnp.pad(p['W_att'][HIDDEN:], ((0, 5), (0, 0)))
    kvg2 = _sc_gather(kv2, eidx, 448).reshape(K, NP_E, 256)
    feats = _attn2(h1, q2, kvg2, lw(l2), sl8, wa, wac8, row(p['b_att']))

    scores8 = jnp.broadcast_to(scores[:, None], (N, 8))
    scorest = jnp.pad(scores[None, :], ((0, 0), (0, NPAD - N)), constant_values=-1.0)
    mask8 = _rank(scores8, scorest)
    up_mask = mask8[:, 0].astype(bool)
    up_pad, down_pad = _sc_partition(mask8[:, 0])
    up_nodes, down_nodes = up_pad[:N_UP], down_pad[:N_DOWN]

    sl128 = jnp.pad(sl8, ((0, 0), (0, 120)))
    cidx = jnp.concatenate([up_pad[:N_UP], down_pad[:N_DOWN],
                            jnp.arange(240, dtype=jnp.int32)])
    udc = _sc_gather(sl128, cidx, 320)
    upc = udc[:N_UP, :8]
    downc = jnp.pad(udc[N_UP:N, :8], ((0, ND_PAD - N_DOWN), (0, 0)))
    upt = jnp.pad(upc.T[:3], ((0, 5), (0, NUP_PAD - N_UP)), constant_values=1e30)
    nidx = _dknn(downc, upt)[:N_DOWN, :M]

    j = up_nodes[nidx.reshape(-1)]
    i = jnp.repeat(down_nodes, M)

    fidx = jnp.pad(i, (0, 38400 - N_DOWN * M))
    fmsg = _sc_gather(feats, fidx, 400)
    jpad = jnp.pad(j, (0, 38400 - N_DOWN * M), constant_values=N)
    agg = jax.ops.segment_max(fmsg, jpad, num_segments=N + 1)[:N]
    agg = jnp.where(jnp.isfinite(agg), agg, 0.0)

    s_l = sl8[:, :3]
    return (agg, up_mask, i, j, s_l)


# speedup vs baseline: 1.2626x; 1.2626x over previous
import functools

import jax
import jax.numpy as jnp
from jax import lax
from jax.experimental import pallas as pl
from jax.experimental.pallas import tpu as pltpu
from jax.experimental.pallas import tpu_sc as plsc

N = 10000
IN_DIM = 128
HIDDEN = 128
HEADS = 8
DH = 16
K = 7
M = 5
N_UP = 2500
N_DOWN = N - N_UP

NPAD = 10112
BQ = 256
BQK = 512
NUP_PAD = 2560
BQ2 = 512
BN = 1000

_INF = float('inf')


def _preamble_body(h_ref, wc_ref, bc_ref, we_ref, be_ref, wq_ref, bq_ref,
                   wkv_ref, bkv_ref, sl_ref, hh_ref, q_ref, kv_ref):
    h = h_ref[...]
    sl_ref[...] = jnp.dot(h, wc_ref[...], preferred_element_type=jnp.float32) + bc_ref[...]
    hh = jnp.dot(h, we_ref[...], preferred_element_type=jnp.float32) + be_ref[...]
    hh_ref[...] = hh
    q_ref[...] = jnp.dot(hh, wq_ref[...], preferred_element_type=jnp.float32) + bq_ref[...]
    kv_ref[...] = jnp.dot(hh, wkv_ref[...], preferred_element_type=jnp.float32) + bkv_ref[...]


def _preamble(h, wc8, bc8, we, be, wq, bq, wkv, bkv):
    grid = (N // BN,)
    row = pl.BlockSpec((BN, IN_DIM), lambda i: (i, 0))
    full = lambda r, c: pl.BlockSpec((r, c), lambda i: (0, 0))
    return pl.pallas_call(
        _preamble_body,
        grid=grid,
        in_specs=[row, full(IN_DIM, 8), full(1, 8), full(IN_DIM, HIDDEN),
                  full(1, HIDDEN), full(HIDDEN, HIDDEN), full(1, HIDDEN),
                  full(HIDDEN, 2 * HIDDEN), full(1, 2 * HIDDEN)],
        out_specs=[pl.BlockSpec((BN, 8), lambda i: (i, 0)),
                   pl.BlockSpec((BN, HIDDEN), lambda i: (i, 0)),
                   pl.BlockSpec((BN, HIDDEN), lambda i: (i, 0)),
                   pl.BlockSpec((BN, 2 * HIDDEN), lambda i: (i, 0))],
        out_shape=[jax.ShapeDtypeStruct((N, 8), jnp.float32),
                   jax.ShapeDtypeStruct((N, HIDDEN), jnp.float32),
                   jax.ShapeDtypeStruct((N, HIDDEN), jnp.float32),
                   jax.ShapeDtypeStruct((N, 2 * HIDDEN), jnp.float32)],
    )(h, wc8, bc8, we, be, wq, bq, wkv, bkv)


W_KNN = 1280
S_KNN = 8
NPAD2 = W_KNN * S_KNN
_PACK = 2048
_PACK_BITS = 11


def _knn_body(sl_ref, slt_ref, t_ref, out_ref):
    pid = pl.program_id(0)
    qx = sl_ref[:, 0:1]
    qy = sl_ref[:, 1:2]
    qz = sl_ref[:, 2:3]
    jcol = lax.broadcasted_iota(jnp.int32, (BQK, W_KNN), 1)
    row = pid * BQK + lax.broadcasted_iota(jnp.int32, (BQK, W_KNN), 0)
    mbest = jnp.full((BQK, W_KNN), _INF, jnp.float32)
    abest = jnp.zeros((BQK, W_KNN), jnp.int32)
    for m in range(S_KNN):
        dx = qx - slt_ref[0:1, m * W_KNN:(m + 1) * W_KNN]
        dy = qy - slt_ref[1:2, m * W_KNN:(m + 1) * W_KNN]
        dz = qz - slt_ref[2:3, m * W_KNN:(m + 1) * W_KNN]
        d = dx * dx + dy * dy + dz * dz
        d = jnp.where(m * W_KNN + jcol == row, _INF, d)
        better = d < mbest
        mbest = jnp.where(better, d, mbest)
        abest = jnp.where(better, m * _PACK + jcol, abest)

    rowc = pid * BQK + lax.broadcasted_iota(jnp.int32, (BQK, 1), 0)
    found = []
    for t in range(K):
        mv = jnp.min(mbest, axis=1, keepdims=True)
        p = jnp.min(jnp.where(mbest == mv, abest, 2 ** 30), axis=1, keepdims=True)
        jstar = jnp.bitwise_and(p, _PACK - 1)
        mstar = lax.shift_right_logical(p, _PACK_BITS)
        gsel = mstar * W_KNN + jstar
        out_ref[:, t:t + 1] = gsel
        found.append(gsel)
        if t == K - 1:
            break
        onehot = jcol == jstar
        cand = jnp.dot(onehot.astype(jnp.float32), t_ref[...],
                       preferred_element_type=jnp.float32)
        ds = []
        for m in range(S_KNN):
            dx = qx - cand[:, 4 * m:4 * m + 1]
            dy = qy - cand[:, 4 * m + 1:4 * m + 2]
            dz = qz - cand[:, 4 * m + 2:4 * m + 3]
            dm = dx * dx + dy * dy + dz * dz
            gc = m * W_KNN + jstar
            bad = gc == rowc
            for f in found:
                bad = bad | (gc == f)
            ds.append(jnp.where(bad, _INF, dm))
        dall = jnp.concatenate(ds, axis=1)
        nm = jnp.min(dall, axis=1, keepdims=True)
        mi = jnp.min(jnp.where(dall == nm,
                               lax.broadcasted_iota(jnp.int32, (BQK, S_KNN), 1),
                               S_KNN), axis=1, keepdims=True)
        na = mi * _PACK + jstar
        mbest = jnp.where(onehot, jnp.broadcast_to(nm, (BQK, W_KNN)), mbest)
        abest = jnp.where(onehot, jnp.broadcast_to(na, (BQK, W_KNN)), abest)
    out_ref[:, K:K + 1] = jnp.zeros((BQK, 1), jnp.int32)


def _knn(sl8, slt, tbl):
    grid = (pl.cdiv(N, BQK),)
    return pl.pallas_call(
        _knn_body,
        grid=grid,
        in_specs=[pl.BlockSpec((BQK, 8), lambda i: (i, 0)),
                  pl.BlockSpec((8, NPAD2), lambda i: (0, 0)),
                  pl.BlockSpec((W_KNN, 4 * S_KNN), lambda i: (0, 0))],
        out_specs=pl.BlockSpec((BQK, 8), lambda i: (i, 0)),
        out_shape=jax.ShapeDtypeStruct((N, 8), jnp.int32),
    )(sl8, slt, tbl)


def _head_ones():
    r = lax.broadcasted_iota(jnp.int32, (HIDDEN, HEADS), 0)
    c = lax.broadcasted_iota(jnp.int32, (HIDDEN, HEADS), 1)
    return (r // DH == c).astype(jnp.float32)


def _ln(x, s, b):
    m = jnp.mean(x, axis=-1, keepdims=True)
    v = jnp.mean((x - m) ** 2, axis=-1, keepdims=True)
    return (x - m) / jnp.sqrt(v + 1e-5) * s + b


def _attn_core(hh_ref, q_ref, kvg_ref, wo_ref, bo_ref, l1s_ref, l1b_ref,
               w1_ref, b1_ref, w2_ref, b2_ref, l2s_ref, l2b_ref):
    oh = _head_ones()
    q = q_ref[...]
    ss = []
    for kk in range(K):
        kblk = kvg_ref[kk, :, 0:128]
        ss.append(jnp.dot(kblk * q, oh, preferred_element_type=jnp.float32) * 0.25)
    smax = ss[0]
    for kk in range(1, K):
        smax = jnp.maximum(smax, ss[kk])
    exs = [jnp.exp(s - smax) for s in ss]
    denom = exs[0]
    for kk in range(1, K):
        denom = denom + exs[kk]
    denom = denom + 1e-6
    wv = None
    for kk in range(K):
        ex128 = jnp.dot(exs[kk], oh.T, preferred_element_type=jnp.float32)
        vblk = kvg_ref[kk, :, 128:256]
        wv = ex128 * vblk if wv is None else wv + ex128 * vblk
    attn = wv / jnp.dot(denom, oh.T, preferred_element_type=jnp.float32)
    h2 = hh_ref[...] + jnp.dot(attn, wo_ref[...], preferred_element_type=jnp.float32) + bo_ref[...]
    h2 = _ln(h2, l1s_ref[...], l1b_ref[...])
    ff = jnp.dot(jax.nn.relu(jnp.dot(h2, w1_ref[...], preferred_element_type=jnp.float32) + b1_ref[...]),
                 w2_ref[...], preferred_element_type=jnp.float32) + b2_ref[...]
    return _ln(h2 + ff, l2s_ref[...], l2b_ref[...])


def _attn1_body(hh_ref, q_ref, kvg_ref, wo_ref, bo_ref, l1s_ref, l1b_ref,
                w1_ref, b1_ref, w2_ref, b2_ref, l2s_ref, l2b_ref,
                wq2_ref, bq2_ref, wkv2_ref, bkv2_ref,
                h3_ref, q2_ref, kv2_ref):
    h3 = _attn_core(hh_ref, q_ref, kvg_ref, wo_ref, bo_ref, l1s_ref, l1b_ref,
                    w1_ref, b1_ref, w2_ref, b2_ref, l2s_ref, l2b_ref)
    h3_ref[...] = h3
    q2_ref[...] = jnp.dot(h3, wq2_ref[...], preferred_element_type=jnp.float32) + bq2_ref[...]
    kv2_ref[...] = jnp.dot(h3, wkv2_ref[...], preferred_element_type=jnp.float32) + bkv2_ref[...]


def _attn2_body(hh_ref, q_ref, kvg_ref, wo_ref, bo_ref, l1s_ref, l1b_ref,
                w1_ref, b1_ref, w2_ref, b2_ref, l2s_ref, l2b_ref,
                sl_ref, wa_ref, wac_ref, ba_ref, feats_ref):
    h3 = _attn_core(hh_ref, q_ref, kvg_ref, wo_ref, bo_ref, l1s_ref, l1b_ref,
                    w1_ref, b1_ref, w2_ref, b2_ref, l2s_ref, l2b_ref)
    feats_ref[...] = (jnp.dot(h3, wa_ref[...], preferred_element_type=jnp.float32)
                      + jnp.dot(sl_ref[...], wac_ref[...], preferred_element_type=jnp.float32)
                      + ba_ref[...])


BNA = 1024
NP_E = 10240


def _attn_specs():
    row128 = pl.BlockSpec((BNA, HIDDEN), lambda i: (i, 0))
    rowkvg = pl.BlockSpec((K, BNA, 256), lambda i: (0, i, 0))
    full = lambda r, c: pl.BlockSpec((r, c), lambda i: (0, 0))
    w = [full(HIDDEN, HIDDEN), full(1, HIDDEN), full(1, HIDDEN), full(1, HIDDEN),
         full(HIDDEN, 2 * HIDDEN), full(1, 2 * HIDDEN), full(2 * HIDDEN, HIDDEN),
         full(1, HIDDEN), full(1, HIDDEN), full(1, HIDDEN)]
    return [row128, row128, rowkvg] + w, full


def _attn1(hh, q, kvg, lw, wq2, bq2, wkv2, bkv2):
    base, full = _attn_specs()
    in_specs = base + [full(HIDDEN, HIDDEN), full(1, HIDDEN),
                       full(HIDDEN, 2 * HIDDEN), full(1, 2 * HIDDEN)]
    row128 = pl.BlockSpec((BNA, HIDDEN), lambda i: (i, 0))
    return pl.pallas_call(
        _attn1_body,
        grid=(NP_E // BNA,),
        in_specs=in_specs,
        out_specs=[row128, row128, pl.BlockSpec((BNA, 2 * HIDDEN), lambda i: (i, 0))],
        out_shape=[jax.ShapeDtypeStruct((N, HIDDEN), jnp.float32),
                   jax.ShapeDtypeStruct((N, HIDDEN), jnp.float32),
                   jax.ShapeDtypeStruct((N, 2 * HIDDEN), jnp.float32)],
    )(hh, q, kvg, *lw, wq2, bq2, wkv2, bkv2)


def _attn2(hh, q, kvg, lw, sl8, wa, wac8, ba):
    base, full = _attn_specs()
    in_specs = base + [pl.BlockSpec((BNA, 8), lambda i: (i, 0)),
                       full(HIDDEN, HIDDEN), full(8, HIDDEN), full(1, HIDDEN)]
    return pl.pallas_call(
        _attn2_body,
        grid=(NP_E // BNA,),
        in_specs=in_specs,
        out_specs=pl.BlockSpec((BNA, HIDDEN), lambda i: (i, 0)),
        out_shape=jax.ShapeDtypeStruct((N, HIDDEN), jnp.float32),
    )(hh, q, kvg, *lw, sl8, wa, wac8, ba)


def _rank_body(s8_ref, st_ref, out_ref):
    pid = pl.program_id(0)
    si = s8_ref[:, 0:1]
    srow = st_ref[0:1, :]
    col = lax.broadcasted_iota(jnp.int32, (BQ, NPAD), 1)
    row = pid * BQ + lax.broadcasted_iota(jnp.int32, (BQ, NPAD), 0)
    gt = (srow > si).astype(jnp.int32)
    eq = ((srow == si) & (col > row)).astype(jnp.int32)
    rank = jnp.sum(gt + eq, axis=1, keepdims=True)
    out_ref[...] = jnp.broadcast_to((rank < N_UP).astype(jnp.int32), (BQ, 8))


def _rank(scores8, scorest):
    return pl.pallas_call(
        _rank_body,
        grid=(pl.cdiv(N, BQ),),
        in_specs=[pl.BlockSpec((BQ, 8), lambda i: (i, 0)),
                  pl.BlockSpec((1, NPAD), lambda i: (0, 0))],
        out_specs=pl.BlockSpec((BQ, 8), lambda i: (i, 0)),
        out_shape=jax.ShapeDtypeStruct((N, 8), jnp.int32),
    )(scores8, scorest)


ND_PAD = 7680


def _dknn_body(dc_ref, upt_ref, out_ref, dist_ref):
    qx = dc_ref[:, 0:1]
    qy = dc_ref[:, 1:2]
    qz = dc_ref[:, 2:3]
    dx = qx - upt_ref[0:1, :]
    dy = qy - upt_ref[1:2, :]
    dz = qz - upt_ref[2:3, :]
    dist_ref[...] = dx * dx + dy * dy + dz * dz
    col = lax.broadcasted_iota(jnp.int32, (BQ2, NUP_PAD), 1)
    for t in range(M):
        d = dist_ref[...]
        m = jnp.min(d, axis=1, keepdims=True)
        idx = jnp.min(jnp.where(d == m, col, NUP_PAD), axis=1, keepdims=True)
        out_ref[:, t:t + 1] = idx
        dist_ref[...] = jnp.where(col == idx, _INF, d)
    out_ref[:, M:M + 3] = jnp.zeros((BQ2, 3), jnp.int32)


def _dknn(downc, upt):
    return pl.pallas_call(
        _dknn_body,
        grid=(ND_PAD // BQ2,),
        in_specs=[pl.BlockSpec((BQ2, 8), lambda i: (i, 0)),
                  pl.BlockSpec((8, NUP_PAD), lambda i: (0, 0))],
        out_specs=pl.BlockSpec((BQ2, 8), lambda i: (i, 0)),
        out_shape=jax.ShapeDtypeStruct((ND_PAD, 8), jnp.int32),
        scratch_shapes=[pltpu.VMEM((BQ2, NUP_PAD), jnp.float32)],
    )(downc, upt)


_NW = 32


def _sc_gather(table, idx, chunk):
    b = idx.shape[0]
    d = table.shape[1]
    b_per_w = b // _NW
    nchunks = b_per_w // chunk
    mesh = plsc.VectorSubcoreMesh(core_axis_name="c", subcore_axis_name="s")

    @functools.partial(
        pl.kernel, mesh=mesh,
        out_type=jax.ShapeDtypeStruct((b, d), jnp.float32),
        scratch_types=[pltpu.VMEM((chunk,), jnp.int32),
                       pltpu.VMEM((chunk, d), jnp.float32),
                       pltpu.SemaphoreType.DMA],
    )
    def k(table_hbm, idx_hbm, out_hbm, idx_v, rows_v, sem):
        wid = lax.axis_index("s") * 2 + lax.axis_index("c")
        base = wid * b_per_w

        def body(ci, _):
            off = base + ci * chunk
            pltpu.sync_copy(idx_hbm.at[pl.ds(off, chunk)], idx_v)
            pltpu.async_copy(table_hbm.at[idx_v], rows_v, sem).wait()
            pltpu.sync_copy(rows_v, out_hbm.at[pl.ds(off, chunk)])
            return 0

        lax.fori_loop(0, nchunks, body, 0)

    return k(table, idx)


def _sc_partition(mask_i32):
    mesh = plsc.VectorSubcoreMesh(core_axis_name="c", subcore_axis_name="s")

    @functools.partial(
        pl.kernel, mesh=mesh,
        compiler_params=pltpu.CompilerParams(needs_layout_passes=False),
        out_type=[jax.ShapeDtypeStruct((2512,), jnp.int32),
                  jax.ShapeDtypeStruct((7504,), jnp.int32)],
        scratch_types=[pltpu.VMEM((N,), jnp.int32),
                       pltpu.VMEM((2512,), jnp.int32),
                       pltpu.VMEM((7504,), jnp.int32)],
    )
    def k(mask_hbm, up_hbm, down_hbm, mask_v, up_v, down_v):
        wid = lax.axis_index("s") * 2 + lax.axis_index("c")

        def compact(target, out_v):
            pltpu.sync_copy(mask_hbm, mask_v)

            def body(ci, off):
                v = mask_v[pl.ds(ci * 16, 16)]
                want = v == jnp.full((16,), target, jnp.int32)
                ids = lax.iota(jnp.int32, 16) + jnp.full((16,), ci * 16, jnp.int32)
                csum = jnp.cumsum(want.astype(jnp.int32))
                pos = csum + jnp.full((16,), off - 1, jnp.int32)
                plsc.store_scatter(out_v, [pos], ids, mask=want)
                return off + jnp.sum(want.astype(jnp.int32))

            lax.fori_loop(0, N // 16, body, jnp.int32(0))

        @pl.when(wid == 0)
        def _():
            compact(1, up_v)
            pltpu.sync_copy(up_v, up_hbm)

        @pl.when(wid == 1)
        def _():
            compact(0, down_v)
            pltpu.sync_copy(down_v, down_hbm)

    return k(mask_i32)


def kernel(h, c, scores, object_, params):
    p = params
    l1, l2 = p['layers'][0], p['layers'][1]

    wc8 = jnp.pad(p['W_coord'], ((0, 0), (0, 5)))
    bc8 = jnp.pad(p['b_coord'], (0, 5)).reshape(1, 8)
    row = lambda x: x.reshape(1, -1)
    wkv1 = jnp.concatenate([l1['Wk'], l1['Wv']], axis=1)
    bkv1 = jnp.concatenate([l1['bk'], l1['bv']]).reshape(1, -1)
    wkv2 = jnp.concatenate([l2['Wk'], l2['Wv']], axis=1)
    bkv2 = jnp.concatenate([l2['bk'], l2['bv']]).reshape(1, -1)

    sl8, hh0, q1, kv1 = _preamble(h, wc8, bc8, p['W_emb'], row(p['b_emb']),
                                  l1['Wq'], row(l1['bq']), wkv1, bkv1)

    slt = jnp.pad(sl8.T[:3], ((0, 5), (0, NPAD2 - N)), constant_values=1e30)
    tbl = jnp.pad(slt[:3].reshape(3, S_KNN, W_KNN).transpose(2, 1, 0),
                  ((0, 0), (0, 0), (0, 1))).reshape(W_KNN, 4 * S_KNN)
    knn = _knn(sl8, slt, tbl)
    src = knn[:, :K]

    lw = lambda l: (l['Wo'], row(l['bo']), row(l['ln1_s']), row(l['ln1_b']),
                    l['W1'], row(l['b1']), l['W2'], row(l['b2']),
                    row(l['ln2_s']), row(l['ln2_b']))

    eidx = jnp.pad(src.T, ((0, 0), (0, NP_E - N))).reshape(-1)

    kvg1 = _sc_gather(kv1, eidx, 448).reshape(K, NP_E, 256)
    h1, q2, kv2 = _attn1(hh0, q1, kvg1, lw(l1), l2['Wq'], row(l2['bq']), wkv2, bkv2)

    wa = p['W_att'][:HIDDEN]
    wac8 = jnp.pad(p['W_att'][HIDDEN:], ((0, 5), (0, 0)))
    kvg2 = _sc_gather(kv2, eidx, 448).reshape(K, NP_E, 256)
    feats = _attn2(h1, q2, kvg2, lw(l2), sl8, wa, wac8, row(p['b_att']))

    scores8 = jnp.broadcast_to(scores[:, None], (N, 8))
    scorest = jnp.pad(scores[None, :], ((0, 0), (0, NPAD - N)), constant_values=-1.0)
    mask8 = _rank(scores8, scorest)
    up_mask = mask8[:, 0].astype(bool)
    up_pad, down_pad = _sc_partition(mask8[:, 0])
    up_nodes, down_nodes = up_pad[:N_UP], down_pad[:N_DOWN]

    sl128 = jnp.pad(sl8, ((0, 0), (0, 120)))
    cidx = jnp.concatenate([up_pad[:N_UP], down_pad[:N_DOWN],
                            jnp.arange(240, dtype=jnp.int32)])
    udc = _sc_gather(sl128, cidx, 320)
    upc = udc[:N_UP, :8]
    downc = jnp.pad(udc[N_UP:N, :8], ((0, ND_PAD - N_DOWN), (0, 0)))
    upt = jnp.pad(upc.T[:3], ((0, 5), (0, NUP_PAD - N_UP)), constant_values=1e30)
    nidx = _dknn(downc, upt)[:N_DOWN, :M]

    j = up_nodes[nidx.reshape(-1)]
    i = jnp.repeat(down_nodes, M)

    fidx = jnp.pad(i, (0, 38400 - N_DOWN * M))
    fmsg = _sc_gather(feats, fidx, 400)
    jpad = jnp.pad(j, (0, 38400 - N_DOWN * M), constant_values=N)
    agg = jax.ops.segment_max(fmsg, jpad, num_segments=N + 1)[:N]
    agg = jnp.where(jnp.isfinite(agg), agg, 0.0)

    s_l = sl8[:, :3]
    return (agg, up_mask, i, j, s_l)


# speedup vs baseline: 1.2953x; 1.0259x over previous
import functools

import jax
import jax.numpy as jnp
from jax import lax
from jax.experimental import pallas as pl
from jax.experimental.pallas import tpu as pltpu
from jax.experimental.pallas import tpu_sc as plsc

N = 10000
IN_DIM = 128
HIDDEN = 128
HEADS = 8
DH = 16
K = 7
M = 5
N_UP = 2500
N_DOWN = N - N_UP

NPAD = 10112
BQ = 256
BQK = 512
NUP_PAD = 2560
BQ2 = 512
BN = 1000

_INF = float('inf')


def _preamble_body(h_ref, wc_ref, bc_ref, we_ref, be_ref, wq_ref, bq_ref,
                   wkv_ref, bkv_ref, sl_ref, hh_ref, q_ref, kv_ref):
    h = h_ref[...]
    sl_ref[...] = jnp.dot(h, wc_ref[...], preferred_element_type=jnp.float32) + bc_ref[...]
    hh = jnp.dot(h, we_ref[...], preferred_element_type=jnp.float32) + be_ref[...]
    hh_ref[...] = hh
    q_ref[...] = jnp.dot(hh, wq_ref[...], preferred_element_type=jnp.float32) + bq_ref[...]
    kv_ref[...] = jnp.dot(hh, wkv_ref[...], preferred_element_type=jnp.float32) + bkv_ref[...]


def _preamble(h, wc8, bc8, we, be, wq, bq, wkv, bkv):
    grid = (N // BN,)
    row = pl.BlockSpec((BN, IN_DIM), lambda i: (i, 0))
    full = lambda r, c: pl.BlockSpec((r, c), lambda i: (0, 0))
    return pl.pallas_call(
        _preamble_body,
        grid=grid,
        in_specs=[row, full(IN_DIM, 8), full(1, 8), full(IN_DIM, HIDDEN),
                  full(1, HIDDEN), full(HIDDEN, HIDDEN), full(1, HIDDEN),
                  full(HIDDEN, 2 * HIDDEN), full(1, 2 * HIDDEN)],
        out_specs=[pl.BlockSpec((BN, 8), lambda i: (i, 0)),
                   pl.BlockSpec((BN, HIDDEN), lambda i: (i, 0)),
                   pl.BlockSpec((BN, HIDDEN), lambda i: (i, 0)),
                   pl.BlockSpec((BN, 2 * HIDDEN), lambda i: (i, 0))],
        out_shape=[jax.ShapeDtypeStruct((N, 8), jnp.float32),
                   jax.ShapeDtypeStruct((N, HIDDEN), jnp.float32),
                   jax.ShapeDtypeStruct((N, HIDDEN), jnp.float32),
                   jax.ShapeDtypeStruct((N, 2 * HIDDEN), jnp.float32)],
    )(h, wc8, bc8, we, be, wq, bq, wkv, bkv)


W_KNN = 1280
S_KNN = 8
NPAD2 = W_KNN * S_KNN
_PACK = 2048
_PACK_BITS = 11


def _knn_body(sl_ref, slt_ref, t_ref, out_ref):
    pid = pl.program_id(0)
    qx = sl_ref[:, 0:1]
    qy = sl_ref[:, 1:2]
    qz = sl_ref[:, 2:3]
    jcol = lax.broadcasted_iota(jnp.int32, (BQK, W_KNN), 1)
    row = pid * BQK + lax.broadcasted_iota(jnp.int32, (BQK, W_KNN), 0)
    mbest = jnp.full((BQK, W_KNN), _INF, jnp.float32)
    abest = jnp.zeros((BQK, W_KNN), jnp.int32)
    for m in range(S_KNN):
        dx = qx - slt_ref[0:1, m * W_KNN:(m + 1) * W_KNN]
        dy = qy - slt_ref[1:2, m * W_KNN:(m + 1) * W_KNN]
        dz = qz - slt_ref[2:3, m * W_KNN:(m + 1) * W_KNN]
        d = dx * dx + dy * dy + dz * dz
        d = jnp.where(m * W_KNN + jcol == row, _INF, d)
        better = d < mbest
        mbest = jnp.where(better, d, mbest)
        abest = jnp.where(better, m * _PACK + jcol, abest)

    rowc = pid * BQK + lax.broadcasted_iota(jnp.int32, (BQK, 1), 0)
    found = []
    for t in range(K):
        mv = jnp.min(mbest, axis=1, keepdims=True)
        p = jnp.min(jnp.where(mbest == mv, abest, 2 ** 30), axis=1, keepdims=True)
        jstar = jnp.bitwise_and(p, _PACK - 1)
        mstar = lax.shift_right_logical(p, _PACK_BITS)
        gsel = mstar * W_KNN + jstar
        out_ref[:, t:t + 1] = gsel
        found.append(gsel)
        if t == K - 1:
            break
        onehot = jcol == jstar
        cand = jnp.dot(onehot.astype(jnp.float32), t_ref[...],
                       preferred_element_type=jnp.float32)
        ds = []
        for m in range(S_KNN):
            dx = qx - cand[:, 4 * m:4 * m + 1]
            dy = qy - cand[:, 4 * m + 1:4 * m + 2]
            dz = qz - cand[:, 4 * m + 2:4 * m + 3]
            dm = dx * dx + dy * dy + dz * dz
            gc = m * W_KNN + jstar
            bad = gc == rowc
            for f in found:
                bad = bad | (gc == f)
            ds.append(jnp.where(bad, _INF, dm))
        dall = jnp.concatenate(ds, axis=1)
        nm = jnp.min(dall, axis=1, keepdims=True)
        mi = jnp.min(jnp.where(dall == nm,
                               lax.broadcasted_iota(jnp.int32, (BQK, S_KNN), 1),
                               S_KNN), axis=1, keepdims=True)
        na = mi * _PACK + jstar
        mbest = jnp.where(onehot, jnp.broadcast_to(nm, (BQK, W_KNN)), mbest)
        abest = jnp.where(onehot, jnp.broadcast_to(na, (BQK, W_KNN)), abest)
    out_ref[:, K:K + 1] = jnp.zeros((BQK, 1), jnp.int32)


def _knn(sl8, slt, tbl):
    grid = (pl.cdiv(N, BQK),)
    return pl.pallas_call(
        _knn_body,
        grid=grid,
        in_specs=[pl.BlockSpec((BQK, 8), lambda i: (i, 0)),
                  pl.BlockSpec((8, NPAD2), lambda i: (0, 0)),
                  pl.BlockSpec((W_KNN, 4 * S_KNN), lambda i: (0, 0))],
        out_specs=pl.BlockSpec((BQK, 8), lambda i: (i, 0)),
        out_shape=jax.ShapeDtypeStruct((N, 8), jnp.int32),
    )(sl8, slt, tbl)


def _head_ones():
    r = lax.broadcasted_iota(jnp.int32, (HIDDEN, HEADS), 0)
    c = lax.broadcasted_iota(jnp.int32, (HIDDEN, HEADS), 1)
    return (r // DH == c).astype(jnp.float32)


def _ln(x, s, b):
    m = jnp.mean(x, axis=-1, keepdims=True)
    v = jnp.mean((x - m) ** 2, axis=-1, keepdims=True)
    return (x - m) / jnp.sqrt(v + 1e-5) * s + b


def _attn_core(hh_ref, q_ref, kvg_ref, wo_ref, bo_ref, l1s_ref, l1b_ref,
               w1_ref, b1_ref, w2_ref, b2_ref, l2s_ref, l2b_ref):
    oh = _head_ones()
    q = q_ref[...]
    ss = []
    for kk in range(K):
        kblk = kvg_ref[kk, :, 0:128]
        ss.append(jnp.dot(kblk * q, oh, preferred_element_type=jnp.float32) * 0.25)
    smax = ss[0]
    for kk in range(1, K):
        smax = jnp.maximum(smax, ss[kk])
    exs = [jnp.exp(s - smax) for s in ss]
    denom = exs[0]
    for kk in range(1, K):
        denom = denom + exs[kk]
    denom = denom + 1e-6
    wv = None
    for kk in range(K):
        ex128 = jnp.dot(exs[kk], oh.T, preferred_element_type=jnp.float32)
        vblk = kvg_ref[kk, :, 128:256]
        wv = ex128 * vblk if wv is None else wv + ex128 * vblk
    attn = wv / jnp.dot(denom, oh.T, preferred_element_type=jnp.float32)
    h2 = hh_ref[...] + jnp.dot(attn, wo_ref[...], preferred_element_type=jnp.float32) + bo_ref[...]
    h2 = _ln(h2, l1s_ref[...], l1b_ref[...])
    ff = jnp.dot(jax.nn.relu(jnp.dot(h2, w1_ref[...], preferred_element_type=jnp.float32) + b1_ref[...]),
                 w2_ref[...], preferred_element_type=jnp.float32) + b2_ref[...]
    return _ln(h2 + ff, l2s_ref[...], l2b_ref[...])


def _attn1_body(hh_ref, q_ref, kvg_ref, wo_ref, bo_ref, l1s_ref, l1b_ref,
                w1_ref, b1_ref, w2_ref, b2_ref, l2s_ref, l2b_ref,
                wq2_ref, bq2_ref, wkv2_ref, bkv2_ref,
                h3_ref, q2_ref, kv2_ref):
    h3 = _attn_core(hh_ref, q_ref, kvg_ref, wo_ref, bo_ref, l1s_ref, l1b_ref,
                    w1_ref, b1_ref, w2_ref, b2_ref, l2s_ref, l2b_ref)
    h3_ref[...] = h3
    q2_ref[...] = jnp.dot(h3, wq2_ref[...], preferred_element_type=jnp.float32) + bq2_ref[...]
    kv2_ref[...] = jnp.dot(h3, wkv2_ref[...], preferred_element_type=jnp.float32) + bkv2_ref[...]


def _attn2_body(hh_ref, q_ref, kvg_ref, wo_ref, bo_ref, l1s_ref, l1b_ref,
                w1_ref, b1_ref, w2_ref, b2_ref, l2s_ref, l2b_ref,
                sl_ref, wa_ref, wac_ref, ba_ref, feats_ref):
    h3 = _attn_core(hh_ref, q_ref, kvg_ref, wo_ref, bo_ref, l1s_ref, l1b_ref,
                    w1_ref, b1_ref, w2_ref, b2_ref, l2s_ref, l2b_ref)
    feats_ref[...] = (jnp.dot(h3, wa_ref[...], preferred_element_type=jnp.float32)
                      + jnp.dot(sl_ref[...], wac_ref[...], preferred_element_type=jnp.float32)
                      + ba_ref[...])


BNA = 1024
NP_E = 10240


def _attn_specs():
    row128 = pl.BlockSpec((BNA, HIDDEN), lambda i: (i, 0))
    rowkvg = pl.BlockSpec((K, BNA, 256), lambda i: (0, i, 0))
    full = lambda r, c: pl.BlockSpec((r, c), lambda i: (0, 0))
    w = [full(HIDDEN, HIDDEN), full(1, HIDDEN), full(1, HIDDEN), full(1, HIDDEN),
         full(HIDDEN, 2 * HIDDEN), full(1, 2 * HIDDEN), full(2 * HIDDEN, HIDDEN),
         full(1, HIDDEN), full(1, HIDDEN), full(1, HIDDEN)]
    return [row128, row128, rowkvg] + w, full


def _attn1(hh, q, kvg, lw, wq2, bq2, wkv2, bkv2):
    base, full = _attn_specs()
    in_specs = base + [full(HIDDEN, HIDDEN), full(1, HIDDEN),
                       full(HIDDEN, 2 * HIDDEN), full(1, 2 * HIDDEN)]
    row128 = pl.BlockSpec((BNA, HIDDEN), lambda i: (i, 0))
    return pl.pallas_call(
        _attn1_body,
        grid=(NP_E // BNA,),
        in_specs=in_specs,
        out_specs=[row128, row128, pl.BlockSpec((BNA, 2 * HIDDEN), lambda i: (i, 0))],
        out_shape=[jax.ShapeDtypeStruct((N, HIDDEN), jnp.float32),
                   jax.ShapeDtypeStruct((N, HIDDEN), jnp.float32),
                   jax.ShapeDtypeStruct((N, 2 * HIDDEN), jnp.float32)],
    )(hh, q, kvg, *lw, wq2, bq2, wkv2, bkv2)


def _attn2(hh, q, kvg, lw, sl8, wa, wac8, ba):
    base, full = _attn_specs()
    in_specs = base + [pl.BlockSpec((BNA, 8), lambda i: (i, 0)),
                       full(HIDDEN, HIDDEN), full(8, HIDDEN), full(1, HIDDEN)]
    return pl.pallas_call(
        _attn2_body,
        grid=(NP_E // BNA,),
        in_specs=in_specs,
        out_specs=pl.BlockSpec((BNA, HIDDEN), lambda i: (i, 0)),
        out_shape=jax.ShapeDtypeStruct((N, HIDDEN), jnp.float32),
    )(hh, q, kvg, *lw, sl8, wa, wac8, ba)


def _rank_body(s8_ref, st_ref, out_ref):
    pid = pl.program_id(0)
    si = s8_ref[:, 0:1]
    srow = st_ref[0:1, :]
    col = lax.broadcasted_iota(jnp.int32, (BQ, NPAD), 1)
    row = pid * BQ + lax.broadcasted_iota(jnp.int32, (BQ, NPAD), 0)
    gt = (srow > si).astype(jnp.int32)
    eq = ((srow == si) & (col > row)).astype(jnp.int32)
    rank = jnp.sum(gt + eq, axis=1, keepdims=True)
    out_ref[...] = jnp.broadcast_to((rank < N_UP).astype(jnp.int32), (BQ, 8))


def _rank(scores8, scorest):
    return pl.pallas_call(
        _rank_body,
        grid=(pl.cdiv(N, BQ),),
        in_specs=[pl.BlockSpec((BQ, 8), lambda i: (i, 0)),
                  pl.BlockSpec((1, NPAD), lambda i: (0, 0))],
        out_specs=pl.BlockSpec((BQ, 8), lambda i: (i, 0)),
        out_shape=jax.ShapeDtypeStruct((N, 8), jnp.int32),
    )(scores8, scorest)


ND_PAD = 7680


def _dknn_body(dc_ref, upt_ref, out_ref, dist_ref):
    qx = dc_ref[:, 0:1]
    qy = dc_ref[:, 1:2]
    qz = dc_ref[:, 2:3]
    dx = qx - upt_ref[0:1, :]
    dy = qy - upt_ref[1:2, :]
    dz = qz - upt_ref[2:3, :]
    dist_ref[...] = dx * dx + dy * dy + dz * dz
    col = lax.broadcasted_iota(jnp.int32, (BQ2, NUP_PAD), 1)
    for t in range(M):
        d = dist_ref[...]
        m = jnp.min(d, axis=1, keepdims=True)
        idx = jnp.min(jnp.where(d == m, col, NUP_PAD), axis=1, keepdims=True)
        out_ref[:, t:t + 1] = idx
        dist_ref[...] = jnp.where(col == idx, _INF, d)
    out_ref[:, M:M + 3] = jnp.zeros((BQ2, 3), jnp.int32)


def _dknn(downc, upt):
    return pl.pallas_call(
        _dknn_body,
        grid=(ND_PAD // BQ2,),
        in_specs=[pl.BlockSpec((BQ2, 8), lambda i: (i, 0)),
                  pl.BlockSpec((8, NUP_PAD), lambda i: (0, 0))],
        out_specs=pl.BlockSpec((BQ2, 8), lambda i: (i, 0)),
        out_shape=jax.ShapeDtypeStruct((ND_PAD, 8), jnp.int32),
        scratch_shapes=[pltpu.VMEM((BQ2, NUP_PAD), jnp.float32)],
    )(downc, upt)


_NW = 32


def _sc_gather(table, idx, chunk):
    b = idx.shape[0]
    d = table.shape[1]
    b_per_w = b // _NW
    nchunks = b_per_w // chunk
    mesh = plsc.VectorSubcoreMesh(core_axis_name="c", subcore_axis_name="s")

    @functools.partial(
        pl.kernel, mesh=mesh,
        out_type=jax.ShapeDtypeStruct((b, d), jnp.float32),
        scratch_types=[pltpu.VMEM((chunk,), jnp.int32),
                       pltpu.VMEM((chunk, d), jnp.float32),
                       pltpu.SemaphoreType.DMA],
    )
    def k(table_hbm, idx_hbm, out_hbm, idx_v, rows_v, sem):
        wid = lax.axis_index("s") * 2 + lax.axis_index("c")
        base = wid * b_per_w

        def body(ci, _):
            off = base + ci * chunk
            pltpu.sync_copy(idx_hbm.at[pl.ds(off, chunk)], idx_v)
            pltpu.async_copy(table_hbm.at[idx_v], rows_v, sem).wait()
            pltpu.sync_copy(rows_v, out_hbm.at[pl.ds(off, chunk)])
            return 0

        lax.fori_loop(0, nchunks, body, 0)

    return k(table, idx)


def _sc_partition(mask_i32):
    mesh = plsc.VectorSubcoreMesh(core_axis_name="c", subcore_axis_name="s")

    @functools.partial(
        pl.kernel, mesh=mesh,
        compiler_params=pltpu.CompilerParams(needs_layout_passes=False),
        out_type=[jax.ShapeDtypeStruct((2512,), jnp.int32),
                  jax.ShapeDtypeStruct((7504,), jnp.int32)],
        scratch_types=[pltpu.VMEM((N,), jnp.int32),
                       pltpu.VMEM((2512,), jnp.int32),
                       pltpu.VMEM((7504,), jnp.int32)],
    )
    def k(mask_hbm, up_hbm, down_hbm, mask_v, up_v, down_v):
        wid = lax.axis_index("s") * 2 + lax.axis_index("c")

        def compact(target, out_v):
            pltpu.sync_copy(mask_hbm, mask_v)

            def body(ci, off):
                v = mask_v[pl.ds(ci * 16, 16)]
                want = v == jnp.full((16,), target, jnp.int32)
                ids = lax.iota(jnp.int32, 16) + jnp.full((16,), ci * 16, jnp.int32)
                csum = jnp.cumsum(want.astype(jnp.int32))
                pos = csum + jnp.full((16,), off - 1, jnp.int32)
                plsc.store_scatter(out_v, [pos], ids, mask=want)
                return off + jnp.sum(want.astype(jnp.int32))

            lax.fori_loop(0, N // 16, body, jnp.int32(0))

        @pl.when(wid == 0)
        def _():
            compact(1, up_v)
            pltpu.sync_copy(up_v, up_hbm)

        @pl.when(wid == 1)
        def _():
            compact(0, down_v)
            pltpu.sync_copy(down_v, down_hbm)

    return k(mask_i32)


def kernel(h, c, scores, object_, params):
    p = params
    l1, l2 = p['layers'][0], p['layers'][1]

    wc8 = jnp.pad(p['W_coord'], ((0, 0), (0, 5)))
    bc8 = jnp.pad(p['b_coord'], (0, 5)).reshape(1, 8)
    row = lambda x: x.reshape(1, -1)
    wkv1 = jnp.concatenate([l1['Wk'], l1['Wv']], axis=1)
    bkv1 = jnp.concatenate([l1['bk'], l1['bv']]).reshape(1, -1)
    wkv2 = jnp.concatenate([l2['Wk'], l2['Wv']], axis=1)
    bkv2 = jnp.concatenate([l2['bk'], l2['bv']]).reshape(1, -1)

    sl8, hh0, q1, kv1 = _preamble(h, wc8, bc8, p['W_emb'], row(p['b_emb']),
                                  l1['Wq'], row(l1['bq']), wkv1, bkv1)

    slt = jnp.pad(sl8.T[:3], ((0, 5), (0, NPAD2 - N)), constant_values=1e30)
    tbl = jnp.pad(slt[:3].reshape(3, S_KNN, W_KNN).transpose(2, 1, 0),
                  ((0, 0), (0, 0), (0, 1))).reshape(W_KNN, 4 * S_KNN)
    knn = _knn(sl8, slt, tbl)
    src = knn[:, :K]

    lw = lambda l: (l['Wo'], row(l['bo']), row(l['ln1_s']), row(l['ln1_b']),
                    l['W1'], row(l['b1']), l['W2'], row(l['b2']),
                    row(l['ln2_s']), row(l['ln2_b']))

    eidx = jnp.pad(src.T, ((0, 0), (0, NP_E - N))).reshape(-1)

    kvg1 = _sc_gather(kv1, eidx, 448).reshape(K, NP_E, 256)
    h1, q2, kv2 = _attn1(hh0, q1, kvg1, lw(l1), l2['Wq'], row(l2['bq']), wkv2, bkv2)

    wa = p['W_att'][:HIDDEN]
    wac8 = jnp.pad(p['W_att'][HIDDEN:], ((0, 5), (0, 0)))
    kvg2 = _sc_gather(kv2, eidx, 448).reshape(K, NP_E, 256)
    feats = _attn2(h1, q2, kvg2, lw(l2), sl8, wa, wac8, row(p['b_att']))

    scores8 = jnp.broadcast_to(scores[:, None], (N, 8))
    scorest = jnp.pad(scores[None, :], ((0, 0), (0, NPAD - N)), constant_values=-1.0)
    mask8 = _rank(scores8, scorest)
    up_mask = mask8[:, 0].astype(bool)
    up_pad, down_pad = _sc_partition(mask8[:, 0])
    up_nodes, down_nodes = up_pad[:N_UP], down_pad[:N_DOWN]

    sl128 = jnp.pad(sl8, ((0, 0), (0, 120)))
    cidx = jnp.concatenate([up_pad[:N_UP], down_pad[:N_DOWN],
                            jnp.arange(240, dtype=jnp.int32)])
    udc = _sc_gather(sl128, cidx, 320)
    upc = udc[:N_UP, :8]
    downc = jnp.pad(udc[N_UP:N, :8], ((0, ND_PAD - N_DOWN), (0, 0)))
    upt = jnp.pad(upc.T[:3], ((0, 5), (0, NUP_PAD - N_UP)), constant_values=1e30)
    nidx = _dknn(downc, upt)[:N_DOWN, :M]

    nmsg = N_DOWN * M
    upt_i = jnp.pad(up_pad[:N_UP], (0, 60), constant_values=N)
    dnt_i = jnp.pad(down_pad[:N_DOWN], (0, 7680 - N_DOWN))
    tid = jnp.broadcast_to(
        jnp.concatenate([upt_i, dnt_i]).astype(jnp.float32)[:, None],
        (10240, 128))
    ediv = jnp.repeat(jnp.arange(N_DOWN, dtype=jnp.int32), M) + 2560
    gidx = jnp.concatenate([
        jnp.pad(nidx.reshape(-1), (0, 38400 - nmsg), constant_values=2512),
        jnp.pad(ediv, (0, 38400 - nmsg))])
    ij = _sc_gather(tid, gidx, 400)[:, 0].astype(jnp.int32)
    jpad = ij[:38400]
    j = jpad[:nmsg]
    i = ij[38400:38400 + nmsg]

    fidx = ij[38400:]
    fmsg = _sc_gather(feats, fidx, 400)
    agg = jax.ops.segment_max(fmsg, jpad, num_segments=N + 1)[:N]
    agg = jnp.where(jnp.isfinite(agg), agg, 0.0)

    s_l = sl8[:, :3]
    return (agg, up_mask, i, j, s_l)
